# baseline (speedup 1.0000x reference)
import functools

import jax
import jax.numpy as jnp
from jax import lax
from jax.experimental import pallas as pl
from jax.experimental.pallas import tpu as pltpu
from jax.experimental.pallas import tpu_sc as plsc

_BLK = 256


def _sc_gather(x, idx_all):
    n, d = x.shape
    m = idx_all.shape[0]
    info = plsc.get_sparse_core_info()
    nw = info.num_cores * info.num_subcores
    per_w = m // nw
    chunk = 128
    nchunks = per_w // chunk
    mesh = plsc.VectorSubcoreMesh(core_axis_name="c", subcore_axis_name="s")

    @functools.partial(
        pl.kernel,
        mesh=mesh,
        out_type=jax.ShapeDtypeStruct((m, d), jnp.float32),
        scratch_types=[
            pltpu.VMEM((per_w,), jnp.int32),
            pltpu.VMEM((per_w, d), jnp.float32),
            pltpu.SemaphoreType.DMA,
        ],
    )
    def k(x_hbm, idx_hbm, out_hbm, idx_v, rows_v, sem):
        wid = lax.axis_index("s") * info.num_cores + lax.axis_index("c")
        base = wid * per_w
        pltpu.sync_copy(idx_hbm.at[pl.ds(base, per_w)], idx_v)
        copies = []
        for j in range(nchunks):
            copies.append(
                pltpu.async_copy(
                    x_hbm.at[idx_v.at[pl.ds(j * chunk, chunk)]],
                    rows_v.at[pl.ds(j * chunk, chunk)],
                    sem,
                )
            )
        for c in copies:
            c.wait()
        pltpu.sync_copy(rows_v, out_hbm.at[pl.ds(base, per_w)])

    return k(x, idx_all)


_WCHUNK = 25
_NQ = 4


def _tc_score_body(et_smem, s_ref, o_ref, et_ref, w_hbm, out_ref,
                   w_vmem, sem, cnt_smem):
    i = pl.program_id(0)
    blk = s_ref.shape[0]
    nrel = w_vmem.shape[0]
    nchunks = nrel // _WCHUNK

    @pl.when(i == 0)
    def _issue():
        cnt_smem[0] = 0
        for c in range(nchunks):
            pltpu.make_async_copy(
                w_hbm.at[pl.ds(c * _WCHUNK, _WCHUNK)],
                w_vmem.at[pl.ds(c * _WCHUNK, _WCHUNK)],
                sem.at[c % _NQ],
            ).start()

    t0 = et_smem[i * blk]
    t1 = et_smem[i * blk + blk - 1]
    need = t1 // _WCHUNK + 1

    def wait_body(c):
        pltpu.make_async_copy(
            w_hbm.at[pl.ds(0, _WCHUNK)],
            w_vmem.at[pl.ds(0, _WCHUNK)],
            sem.at[lax.rem(c, _NQ)],
        ).wait()
        return c + 1

    cnt_smem[0] = lax.while_loop(lambda c: c < need, wait_body, cnt_smem[0])

    s = s_ref[...]
    o = o_ref[...]
    et = et_ref[...]
    etb = jnp.broadcast_to(et, s.shape)
    sb = s.astype(jnp.bfloat16)

    nrel = w_vmem.shape[0]
    unroll = 4

    def body(k, acc):
        tb = t0 + k * unroll
        for j in range(unroll):
            t = tb + j
            tc = jnp.minimum(t, nrel - 1)
            r = w_vmem[tc].astype(jnp.bfloat16)
            y = jnp.dot(sb, r, preferred_element_type=jnp.float32)
            acc = jnp.where(etb == t, y, acc)
        return acc

    ntrip = (t1 - t0) // unroll + 1
    acc = lax.fori_loop(0, ntrip, body, jnp.zeros_like(s))
    ns = jnp.sum(s * s, axis=1, keepdims=True)
    no = jnp.sum(o * o, axis=1, keepdims=True)
    out_ref[...] = jnp.sum(acc * o, axis=1, keepdims=True) * lax.rsqrt(ns * no)


def _tc_score(s_rows, o_rows, et_sorted, et_tiled, weights):
    m, d = s_rows.shape
    nrel = weights.shape[0]
    nblk = m // _BLK
    grid_spec = pltpu.PrefetchScalarGridSpec(
        num_scalar_prefetch=1,
        grid=(nblk,),
        in_specs=[
            pl.BlockSpec((_BLK, d), lambda i, et: (i, 0)),
            pl.BlockSpec((_BLK, d), lambda i, et: (i, 0)),
            pl.BlockSpec((_BLK, 1), lambda i, et: (i, 0)),
            pl.BlockSpec(memory_space=pltpu.HBM),
        ],
        out_specs=pl.BlockSpec((_BLK, 1), lambda i, et: (i, 0)),
        scratch_shapes=[
            pltpu.VMEM((nrel, d, d), jnp.float32),
            pltpu.SemaphoreType.DMA((_NQ,)),
            pltpu.SMEM((1,), jnp.int32),
        ],
    )
    return pl.pallas_call(
        _tc_score_body,
        grid_spec=grid_spec,
        out_shape=jax.ShapeDtypeStruct((m, 1), jnp.float32),
    )(et_sorted, s_rows, o_rows, et_tiled, weights)


def kernel(x, edge_index, edge_type, weights):
    m = edge_type.shape[0]
    d = x.shape[1]
    perm = jnp.argsort(edge_type)
    inv_perm = jnp.argsort(perm)
    et_sorted = edge_type[perm].astype(jnp.int32)
    idx_all = jnp.concatenate(
        [edge_index[0][perm], edge_index[1][perm]]
    ).astype(jnp.int32)

    rows = _sc_gather(x, idx_all)
    s_rows = rows[:m]
    o_rows = rows[m:]

    et_col = et_sorted[:, None]
    scores = _tc_score(s_rows, o_rows, et_sorted, et_col, weights)
    return scores[:, 0][inv_perm]


# speedup vs baseline: 1.0590x; 1.0590x over previous
import functools

import jax
import jax.numpy as jnp
from jax import lax
from jax.experimental import pallas as pl
from jax.experimental.pallas import tpu as pltpu
from jax.experimental.pallas import tpu_sc as plsc

_BLK = 256


def _sc_gather(x, idx_all):
    n, d = x.shape
    m = idx_all.shape[0]
    info = plsc.get_sparse_core_info()
    nw = info.num_cores * info.num_subcores
    per_w = m // nw
    chunk = 128
    nchunks = per_w // chunk
    mesh = plsc.VectorSubcoreMesh(core_axis_name="c", subcore_axis_name="s")

    @functools.partial(
        pl.kernel,
        mesh=mesh,
        out_type=jax.ShapeDtypeStruct((m, d), jnp.float32),
        scratch_types=[
            pltpu.VMEM((per_w,), jnp.int32),
            pltpu.VMEM((per_w, d), jnp.float32),
            pltpu.SemaphoreType.DMA,
        ],
    )
    def k(x_hbm, idx_hbm, out_hbm, idx_v, rows_v, sem):
        wid = lax.axis_index("s") * info.num_cores + lax.axis_index("c")
        base = wid * per_w
        pltpu.sync_copy(idx_hbm.at[pl.ds(base, per_w)], idx_v)
        copies = []
        for j in range(nchunks):
            copies.append(
                pltpu.async_copy(
                    x_hbm.at[idx_v.at[pl.ds(j * chunk, chunk)]],
                    rows_v.at[pl.ds(j * chunk, chunk)],
                    sem,
                )
            )
        for c in copies:
            c.wait()
        pltpu.sync_copy(rows_v, out_hbm.at[pl.ds(base, per_w)])

    return k(x, idx_all)


_WCHUNK = 25
_NQ = 4


def _tc_score_body(et_smem, s_ref, o_ref, et_ref, w_hbm, out_ref,
                   w_vmem, sem, cnt_smem):
    i = pl.program_id(0)
    blk = s_ref.shape[0]
    nrel = w_vmem.shape[0]
    nchunks = nrel // _WCHUNK

    @pl.when(i == 0)
    def _issue():
        cnt_smem[0] = 0
        for c in range(nchunks):
            pltpu.make_async_copy(
                w_hbm.at[pl.ds(c * _WCHUNK, _WCHUNK)],
                w_vmem.at[pl.ds(c * _WCHUNK, _WCHUNK)],
                sem.at[c % _NQ],
            ).start()

    t0 = et_smem[i * blk]
    t1 = et_smem[i * blk + blk - 1]
    need = t1 // _WCHUNK + 1

    def wait_body(c):
        pltpu.make_async_copy(
            w_hbm.at[pl.ds(0, _WCHUNK)],
            w_vmem.at[pl.ds(0, _WCHUNK)],
            sem.at[lax.rem(c, _NQ)],
        ).wait()
        return c + 1

    cnt_smem[0] = lax.while_loop(lambda c: c < need, wait_body, cnt_smem[0])

    s = s_ref[...]
    o = o_ref[...]
    et = et_ref[...]
    etb = jnp.broadcast_to(et, s.shape)
    sb = s.astype(jnp.bfloat16)

    nrel = w_vmem.shape[0]
    unroll = 4

    def body(k, acc):
        tb = t0 + k * unroll
        for j in range(unroll):
            t = tb + j
            tc = jnp.minimum(t, nrel - 1)
            r = w_vmem[tc].astype(jnp.bfloat16)
            y = jnp.dot(sb, r, preferred_element_type=jnp.float32)
            acc = jnp.where(etb == t, y, acc)
        return acc

    ntrip = (t1 - t0) // unroll + 1
    acc = lax.fori_loop(0, ntrip, body, jnp.zeros_like(s))
    ns = jnp.sum(s * s, axis=1, keepdims=True)
    no = jnp.sum(o * o, axis=1, keepdims=True)
    out_ref[...] = jnp.sum(acc * o, axis=1, keepdims=True) * lax.rsqrt(ns * no)


def _tc_score(rows, et_sorted, et_col, weights):
    m2, d = rows.shape
    m = m2 // 2
    nrel = weights.shape[0]
    nblk = m // _BLK
    grid_spec = pltpu.PrefetchScalarGridSpec(
        num_scalar_prefetch=1,
        grid=(nblk,),
        in_specs=[
            pl.BlockSpec((_BLK, d), lambda i, et: (i, 0)),
            pl.BlockSpec((_BLK, d), lambda i, et, n=nblk: (i + n, 0)),
            pl.BlockSpec((_BLK, 1), lambda i, et: (i, 0)),
            pl.BlockSpec(memory_space=pltpu.HBM),
        ],
        out_specs=pl.BlockSpec((_BLK, 1), lambda i, et: (i, 0)),
        scratch_shapes=[
            pltpu.VMEM((nrel, d, d), jnp.float32),
            pltpu.SemaphoreType.DMA((_NQ,)),
            pltpu.SMEM((1,), jnp.int32),
        ],
    )
    return pl.pallas_call(
        _tc_score_body,
        grid_spec=grid_spec,
        out_shape=jax.ShapeDtypeStruct((m, 1), jnp.float32),
    )(et_sorted, rows, rows, et_col, weights)


def kernel(x, edge_index, edge_type, weights):
    m = edge_type.shape[0]
    d = x.shape[1]
    perm = jnp.argsort(edge_type)
    inv_perm = jnp.argsort(perm)
    et_sorted = edge_type[perm].astype(jnp.int32)
    idx_all = jnp.concatenate(
        [edge_index[0][perm], edge_index[1][perm]]
    ).astype(jnp.int32)

    rows = _sc_gather(x, idx_all)

    et_col = et_sorted[:, None]
    scores = _tc_score(rows, et_sorted, et_col, weights)
    return scores[:, 0][inv_perm]


# speedup vs baseline: 1.1482x; 1.0842x over previous
import functools

import jax
import jax.numpy as jnp
from jax import lax
from jax.experimental import pallas as pl
from jax.experimental.pallas import tpu as pltpu
from jax.experimental.pallas import tpu_sc as plsc

_BLK = 256


def _sc_gather(x, idx_all):
    n, d = x.shape
    m = idx_all.shape[0]
    info = plsc.get_sparse_core_info()
    nw = info.num_cores * info.num_subcores
    per_w = m // nw
    chunk = 128
    nchunks = per_w // chunk
    mesh = plsc.VectorSubcoreMesh(core_axis_name="c", subcore_axis_name="s")

    @functools.partial(
        pl.kernel,
        mesh=mesh,
        out_type=jax.ShapeDtypeStruct((m, d), jnp.float32),
        scratch_types=[
            pltpu.VMEM((per_w,), jnp.int32),
            pltpu.VMEM((per_w, d), jnp.float32),
            pltpu.SemaphoreType.DMA,
        ],
    )
    def k(x_hbm, idx_hbm, out_hbm, idx_v, rows_v, sem):
        wid = lax.axis_index("s") * info.num_cores + lax.axis_index("c")
        base = wid * per_w
        pltpu.sync_copy(idx_hbm.at[pl.ds(base, per_w)], idx_v)
        copies = []
        for j in range(nchunks):
            copies.append(
                pltpu.async_copy(
                    x_hbm.at[idx_v.at[pl.ds(j * chunk, chunk)]],
                    rows_v.at[pl.ds(j * chunk, chunk)],
                    sem,
                )
            )
        for c in copies:
            c.wait()
        pltpu.sync_copy(rows_v, out_hbm.at[pl.ds(base, per_w)])

    return k(x, idx_all)


_WCHUNK = 25
_NQ = 4


def _tc_score_body(et_smem, s_ref, o_ref, et_ref, w_hbm, out_ref,
                   w_vmem, sem, cnt_smem):
    i = pl.program_id(0)
    blk = s_ref.shape[0]
    nrel = w_vmem.shape[0]
    nchunks = nrel // _WCHUNK

    @pl.when(i == 0)
    def _issue():
        cnt_smem[0] = 0
        for c in range(nchunks):
            pltpu.make_async_copy(
                w_hbm.at[pl.ds(c * _WCHUNK, _WCHUNK)],
                w_vmem.at[pl.ds(c * _WCHUNK, _WCHUNK)],
                sem.at[c % _NQ],
            ).start()

    t0 = et_smem[i * blk]
    t1 = et_smem[i * blk + blk - 1]
    need = t1 // _WCHUNK + 1

    def wait_body(c):
        pltpu.make_async_copy(
            w_hbm.at[pl.ds(0, _WCHUNK)],
            w_vmem.at[pl.ds(0, _WCHUNK)],
            sem.at[lax.rem(c, _NQ)],
        ).wait()
        return c + 1

    cnt_smem[0] = lax.while_loop(lambda c: c < need, wait_body, cnt_smem[0])

    s = s_ref[...]
    o = o_ref[...]
    et = et_ref[...]
    etb = jnp.broadcast_to(et, s.shape)
    sb = s.astype(jnp.bfloat16)

    nrel = w_vmem.shape[0]
    unroll = 8

    def body(k, acc):
        tb = t0 + k * unroll
        for j in range(unroll):
            t = tb + j
            tc = jnp.minimum(t, nrel - 1)
            r = w_vmem[tc].astype(jnp.bfloat16)
            y = jnp.dot(sb, r, preferred_element_type=jnp.float32)
            acc = jnp.where(etb == t, y, acc)
        return acc

    ntrip = (t1 - t0) // unroll + 1
    acc = lax.fori_loop(0, ntrip, body, jnp.zeros_like(s))
    ns = jnp.sum(s * s, axis=1, keepdims=True)
    no = jnp.sum(o * o, axis=1, keepdims=True)
    out_ref[...] = jnp.sum(acc * o, axis=1, keepdims=True) * lax.rsqrt(ns * no)


def _tc_score(rows, et_sorted, et_col, weights):
    m2, d = rows.shape
    m = m2 // 2
    nrel = weights.shape[0]
    nblk = m // _BLK
    grid_spec = pltpu.PrefetchScalarGridSpec(
        num_scalar_prefetch=1,
        grid=(nblk,),
        in_specs=[
            pl.BlockSpec((_BLK, d), lambda i, et: (i, 0)),
            pl.BlockSpec((_BLK, d), lambda i, et, n=nblk: (i + n, 0)),
            pl.BlockSpec((_BLK, 1), lambda i, et: (i, 0)),
            pl.BlockSpec(memory_space=pltpu.HBM),
        ],
        out_specs=pl.BlockSpec((_BLK, 1), lambda i, et: (i, 0)),
        scratch_shapes=[
            pltpu.VMEM((nrel, d, d), jnp.float32),
            pltpu.SemaphoreType.DMA((_NQ,)),
            pltpu.SMEM((1,), jnp.int32),
        ],
    )
    return pl.pallas_call(
        _tc_score_body,
        grid_spec=grid_spec,
        out_shape=jax.ShapeDtypeStruct((m, 1), jnp.float32),
    )(et_sorted, rows, rows, et_col, weights)


def kernel(x, edge_index, edge_type, weights):
    m = edge_type.shape[0]
    d = x.shape[1]
    perm = jnp.argsort(edge_type)
    inv_perm = jnp.argsort(perm)
    et_sorted = edge_type[perm].astype(jnp.int32)
    idx_all = jnp.concatenate(
        [edge_index[0][perm], edge_index[1][perm]]
    ).astype(jnp.int32)

    rows = _sc_gather(x, idx_all)

    et_col = et_sorted[:, None]
    scores = _tc_score(rows, et_sorted, et_col, weights)
    return scores[:, 0][inv_perm]


# speedup vs baseline: 1.1623x; 1.0122x over previous
import functools

import jax
import jax.numpy as jnp
from jax import lax
from jax.experimental import pallas as pl
from jax.experimental.pallas import tpu as pltpu
from jax.experimental.pallas import tpu_sc as plsc

_BLK = 256


def _sc_gather(x, idx_all):
    n, d = x.shape
    m = idx_all.shape[0]
    info = plsc.get_sparse_core_info()
    nw = info.num_cores * info.num_subcores
    per_w = m // nw
    chunk = 128
    nchunks = per_w // chunk
    mesh = plsc.VectorSubcoreMesh(core_axis_name="c", subcore_axis_name="s")

    @functools.partial(
        pl.kernel,
        mesh=mesh,
        out_type=jax.ShapeDtypeStruct((m, d), jnp.float32),
        scratch_types=[
            pltpu.VMEM((per_w,), jnp.int32),
            pltpu.VMEM((per_w, d), jnp.float32),
            pltpu.SemaphoreType.DMA,
        ],
    )
    def k(x_hbm, idx_hbm, out_hbm, idx_v, rows_v, sem):
        wid = lax.axis_index("s") * info.num_cores + lax.axis_index("c")
        base = wid * per_w
        pltpu.sync_copy(idx_hbm.at[pl.ds(base, per_w)], idx_v)
        copies = []
        for j in range(nchunks):
            copies.append(
                pltpu.async_copy(
                    x_hbm.at[idx_v.at[pl.ds(j * chunk, chunk)]],
                    rows_v.at[pl.ds(j * chunk, chunk)],
                    sem,
                )
            )
        for c in copies:
            c.wait()
        pltpu.sync_copy(rows_v, out_hbm.at[pl.ds(base, per_w)])

    return k(x, idx_all)


_WCHUNK = 25
_NQ = 4


def _tc_score_body(et_smem, s_ref, o_ref, et_ref, w_hbm, out_ref,
                   w_vmem, sem, cnt_smem):
    i = pl.program_id(0)
    blk = s_ref.shape[0]
    nrel = w_vmem.shape[0]
    nchunks = nrel // _WCHUNK

    @pl.when(i == 0)
    def _issue():
        cnt_smem[0] = 0
        for c in range(nchunks):
            pltpu.make_async_copy(
                w_hbm.at[pl.ds(c * _WCHUNK, _WCHUNK)],
                w_vmem.at[pl.ds(c * _WCHUNK, _WCHUNK)],
                sem.at[c % _NQ],
            ).start()

    t0 = et_smem[i * blk]
    t1 = et_smem[i * blk + blk - 1]
    need = t1 // _WCHUNK + 1

    def wait_body(c):
        pltpu.make_async_copy(
            w_hbm.at[pl.ds(0, _WCHUNK)],
            w_vmem.at[pl.ds(0, _WCHUNK)],
            sem.at[lax.rem(c, _NQ)],
        ).wait()
        return c + 1

    cnt_smem[0] = lax.while_loop(lambda c: c < need, wait_body, cnt_smem[0])

    s = s_ref[...]
    o = o_ref[...]
    et = et_ref[...]
    etb = jnp.broadcast_to(et, s.shape)
    sb = s.astype(jnp.bfloat16)

    nrel = w_vmem.shape[0]
    unroll = 16

    def body(k, acc):
        tb = t0 + k * unroll
        for j in range(unroll):
            t = tb + j
            tc = jnp.minimum(t, nrel - 1)
            r = w_vmem[tc].astype(jnp.bfloat16)
            y = jnp.dot(sb, r, preferred_element_type=jnp.float32)
            acc = jnp.where(etb == t, y, acc)
        return acc

    ntrip = (t1 - t0) // unroll + 1
    acc = lax.fori_loop(0, ntrip, body, jnp.zeros_like(s))
    ns = jnp.sum(s * s, axis=1, keepdims=True)
    no = jnp.sum(o * o, axis=1, keepdims=True)
    out_ref[...] = jnp.sum(acc * o, axis=1, keepdims=True) * lax.rsqrt(ns * no)


def _tc_score(rows, et_sorted, et_col, weights):
    m2, d = rows.shape
    m = m2 // 2
    nrel = weights.shape[0]
    nblk = m // _BLK
    grid_spec = pltpu.PrefetchScalarGridSpec(
        num_scalar_prefetch=1,
        grid=(nblk,),
        in_specs=[
            pl.BlockSpec((_BLK, d), lambda i, et: (i, 0)),
            pl.BlockSpec((_BLK, d), lambda i, et, n=nblk: (i + n, 0)),
            pl.BlockSpec((_BLK, 1), lambda i, et: (i, 0)),
            pl.BlockSpec(memory_space=pltpu.HBM),
        ],
        out_specs=pl.BlockSpec((_BLK, 1), lambda i, et: (i, 0)),
        scratch_shapes=[
            pltpu.VMEM((nrel, d, d), jnp.float32),
            pltpu.SemaphoreType.DMA((_NQ,)),
            pltpu.SMEM((1,), jnp.int32),
        ],
    )
    return pl.pallas_call(
        _tc_score_body,
        grid_spec=grid_spec,
        out_shape=jax.ShapeDtypeStruct((m, 1), jnp.float32),
    )(et_sorted, rows, rows, et_col, weights)


def kernel(x, edge_index, edge_type, weights):
    m = edge_type.shape[0]
    d = x.shape[1]
    perm = jnp.argsort(edge_type)
    inv_perm = jnp.argsort(perm)
    et_sorted = edge_type[perm].astype(jnp.int32)
    idx_all = jnp.concatenate(
        [edge_index[0][perm], edge_index[1][perm]]
    ).astype(jnp.int32)

    rows = _sc_gather(x, idx_all)

    et_col = et_sorted[:, None]
    scores = _tc_score(rows, et_sorted, et_col, weights)
    return scores[:, 0][inv_perm]
